# Initial kernel scaffold; baseline (speedup 1.0000x reference)
import jax
import jax.numpy as jnp
from jax.experimental import pallas as pl


def kernel(x, table):
    raise NotImplementedError("write your pallas kernel here")


# speedup vs baseline: 4.6544x; 4.6544x over previous
import functools

import numpy as np
import jax
import jax.numpy as jnp
from jax import lax
from jax.experimental import pallas as pl
from jax.experimental.pallas import tpu as pltpu
from jax.experimental.pallas import tpu_sc as plsc

DIM_MODEL = 768
SEQ_LEN = 2048
BATCH = 4
N_ROWS = BATCH * SEQ_LEN

NUM_WORKERS = 32
PER_WORKER = N_ROWS // NUM_WORKERS
CHUNK = 64
LANES = 16
VECS_PER_ROW = DIM_MODEL // LANES


def _pos_encoding_np() -> np.ndarray:
    position = np.arange(0, SEQ_LEN, dtype=np.float32)[:, None]
    s2i = np.arange(0, DIM_MODEL, 2, dtype=np.float32)
    with np.errstate(over="ignore"):
        denom = np.power(np.float32(10000.0), s2i, dtype=np.float32)
    ratio = (position / denom).astype(np.float32)
    enc = np.zeros((SEQ_LEN, DIM_MODEL), dtype=np.float32)
    enc[:, 0::2] = np.sin(ratio)
    enc[:, 1::2] = np.cos(ratio)
    return enc


_POS_ENC = _pos_encoding_np()

_MESH = plsc.VectorSubcoreMesh(core_axis_name="c", subcore_axis_name="s")


@functools.partial(
    pl.kernel,
    mesh=_MESH,
    out_type=jax.ShapeDtypeStruct((N_ROWS, DIM_MODEL), jnp.float32),
    scratch_types=[
        pltpu.VMEM((PER_WORKER,), jnp.int32),
        pltpu.VMEM((CHUNK, DIM_MODEL), jnp.float32),
        pltpu.VMEM((CHUNK, DIM_MODEL), jnp.float32),
        pltpu.SemaphoreType.DMA,
    ],
)
def _embed_sc(idx_hbm, table_hbm, pos_hbm, out_hbm, idx_v, rows_v, pos_v, sem):
    wid = lax.axis_index("s") * 2 + lax.axis_index("c")
    base = wid * PER_WORKER
    pltpu.sync_copy(idx_hbm.at[pl.ds(base, PER_WORKER)], idx_v)

    for ci in range(PER_WORKER // CHUNK):
        off = base + ci * CHUNK
        gather = pltpu.async_copy(
            table_hbm.at[idx_v.at[pl.ds(ci * CHUNK, CHUNK)]], rows_v, sem
        )
        pltpu.sync_copy(pos_hbm.at[pl.ds(lax.rem(off, SEQ_LEN), CHUNK)], pos_v)
        gather.wait()

        def add_row(r, _):
            for j in range(VECS_PER_ROW):
                plsc.addupdate(
                    rows_v.at[r, pl.ds(j * LANES, LANES)],
                    pos_v[r, pl.ds(j * LANES, LANES)],
                )
            return _

        lax.fori_loop(0, CHUNK, add_row, None)
        pltpu.sync_copy(rows_v, out_hbm.at[pl.ds(off, CHUNK)])


def kernel(x, table):
    idx = x.reshape(-1).astype(jnp.int32)
    out = _embed_sc(idx, table, _POS_ENC)
    return out.reshape(BATCH, SEQ_LEN, DIM_MODEL)


# speedup vs baseline: 6.2125x; 1.3348x over previous
import functools

import numpy as np
import jax
import jax.numpy as jnp
from jax import lax
from jax.experimental import pallas as pl
from jax.experimental.pallas import tpu as pltpu
from jax.experimental.pallas import tpu_sc as plsc

DIM_MODEL = 768
SEQ_LEN = 2048
BATCH = 4
N_ROWS = BATCH * SEQ_LEN

NUM_WORKERS = 32
POS_PER_W = SEQ_LEN // NUM_WORKERS
CHUNK = 32
NCHUNKS = 2 * BATCH
LANES = 16
VECS_PER_ROW = DIM_MODEL // LANES


def _pos_encoding_np() -> np.ndarray:
    position = np.arange(0, SEQ_LEN, dtype=np.float32)[:, None]
    s2i = np.arange(0, DIM_MODEL, 2, dtype=np.float32)
    with np.errstate(over="ignore"):
        denom = np.power(np.float32(10000.0), s2i, dtype=np.float32)
    ratio = (position / denom).astype(np.float32)
    enc = np.zeros((SEQ_LEN, DIM_MODEL), dtype=np.float32)
    enc[:, 0::2] = np.sin(ratio)
    enc[:, 1::2] = np.cos(ratio)
    return enc


_POS_ENC = _pos_encoding_np()

_MESH = plsc.VectorSubcoreMesh(core_axis_name="c", subcore_axis_name="s")


@functools.partial(
    pl.kernel,
    mesh=_MESH,
    out_type=jax.ShapeDtypeStruct((N_ROWS, DIM_MODEL), jnp.float32),
    scratch_types=[
        pltpu.VMEM((BATCH * POS_PER_W,), jnp.int32),
        pltpu.VMEM((CHUNK, DIM_MODEL), jnp.float32),
        pltpu.VMEM((CHUNK, DIM_MODEL), jnp.float32),
        pltpu.VMEM((CHUNK, DIM_MODEL), jnp.float32),
        pltpu.VMEM((CHUNK, DIM_MODEL), jnp.float32),
        pltpu.SemaphoreType.DMA,
        pltpu.SemaphoreType.DMA,
        pltpu.SemaphoreType.DMA,
        pltpu.SemaphoreType.DMA,
        pltpu.SemaphoreType.DMA,
        pltpu.SemaphoreType.DMA,
        pltpu.SemaphoreType.DMA,
    ],
)
def _embed_sc(
    x_hbm, table_hbm, pos_hbm, out_hbm,
    idx_v, rows0, rows1, pos0, pos1,
    isem, g0, g1, p0, p1, w0, w1,
):
    rows = (rows0, rows1)
    posb = (pos0, pos1)
    gsem = (g0, g1)
    psem = (p0, p1)
    wsem = (w0, w1)

    wid = lax.axis_index("s") * 2 + lax.axis_index("c")
    pbase = wid * POS_PER_W

    idesc = [
        pltpu.async_copy(
            x_hbm.at[b, pl.ds(pbase, POS_PER_W)],
            idx_v.at[pl.ds(b * POS_PER_W, POS_PER_W)],
            isem,
        )
        for b in range(BATCH)
    ]
    pdesc = [
        pltpu.async_copy(pos_hbm.at[pl.ds(pbase + h * CHUNK, CHUNK)], posb[h], psem[h])
        for h in (0, 1)
    ]
    for d in idesc:
        d.wait()

    def start_gather(ci):
        h, b = divmod(ci, BATCH)
        return pltpu.async_copy(
            table_hbm.at[idx_v.at[pl.ds(b * POS_PER_W + h * CHUNK, CHUNK)]],
            rows[ci & 1],
            gsem[ci & 1],
        )

    gd = {0: start_gather(0)}
    wd = {}
    for ci in range(NCHUNKS):
        h, b = divmod(ci, BATCH)
        buf = ci & 1
        if ci + 1 < NCHUNKS:
            if ci >= 1:
                wd[ci - 1].wait()
            gd[ci + 1] = start_gather(ci + 1)
        gd[ci].wait()
        if ci == h * BATCH:
            pdesc[h].wait()
        rv = rows[buf]
        pv = posb[h]

        def add_row(r, _):
            for j in range(VECS_PER_ROW):
                plsc.addupdate(
                    rv.at[r, pl.ds(j * LANES, LANES)],
                    pv[r, pl.ds(j * LANES, LANES)],
                )
            return _

        lax.fori_loop(0, CHUNK, add_row, None)
        wd[ci] = pltpu.async_copy(
            rv,
            out_hbm.at[pl.ds(b * SEQ_LEN + pbase + h * CHUNK, CHUNK)],
            wsem[buf],
        )
    wd[NCHUNKS - 2].wait()
    wd[NCHUNKS - 1].wait()


def kernel(x, table):
    xi = x.astype(jnp.int32)
    out = _embed_sc(xi, table, _POS_ENC)
    return out.reshape(BATCH, SEQ_LEN, DIM_MODEL)


# speedup vs baseline: 7.6742x; 1.2353x over previous
import functools

import numpy as np
import jax
import jax.numpy as jnp
from jax import lax
from jax.experimental import pallas as pl
from jax.experimental.pallas import tpu as pltpu
from jax.experimental.pallas import tpu_sc as plsc

DIM_MODEL = 768
SEQ_LEN = 2048
BATCH = 4
N_ROWS = BATCH * SEQ_LEN

NUM_WORKERS = 32
POS_PER_W = SEQ_LEN // NUM_WORKERS
CHUNK = POS_PER_W
NCHUNKS = BATCH
LANES = 16
VECS_PER_ROW = DIM_MODEL // LANES


def _pos_head_np() -> np.ndarray:
    position = np.arange(0, SEQ_LEN, dtype=np.float32)[:, None]
    s2i = np.arange(0, DIM_MODEL, 2, dtype=np.float32)
    with np.errstate(over="ignore"):
        denom = np.power(np.float32(10000.0), s2i, dtype=np.float32)
    ratio = (position / denom).astype(np.float32)
    enc = np.zeros((SEQ_LEN, DIM_MODEL), dtype=np.float32)
    enc[:, 0::2] = np.sin(ratio)
    enc[:, 1::2] = np.cos(ratio)
    assert np.all(enc[:, LANES:] == np.tile(np.float32([0.0, 1.0]), DIM_MODEL // 2)[LANES:])
    return np.ascontiguousarray(enc[:, :LANES])


_POS_HEAD = _pos_head_np()

_MESH = plsc.VectorSubcoreMesh(core_axis_name="c", subcore_axis_name="s")


@functools.partial(
    pl.kernel,
    mesh=_MESH,
    out_type=jax.ShapeDtypeStruct((N_ROWS, DIM_MODEL), jnp.float32),
    scratch_types=[
        pltpu.VMEM((BATCH * POS_PER_W,), jnp.int32),
        pltpu.VMEM((POS_PER_W, LANES), jnp.float32),
        pltpu.VMEM((CHUNK, DIM_MODEL), jnp.float32),
        pltpu.VMEM((CHUNK, DIM_MODEL), jnp.float32),
        pltpu.SemaphoreType.DMA,
        pltpu.SemaphoreType.DMA,
        pltpu.SemaphoreType.DMA,
        pltpu.SemaphoreType.DMA,
        pltpu.SemaphoreType.DMA,
        pltpu.SemaphoreType.DMA,
    ],
)
def _embed_sc(
    x_hbm, table_hbm, pos_hbm, out_hbm,
    idx_v, pos_v, rows0, rows1,
    isem, psem, g0, g1, w0, w1,
):
    rows = (rows0, rows1)
    gsem = (g0, g1)
    wsem = (w0, w1)

    wid = lax.axis_index("s") * 2 + lax.axis_index("c")
    pbase = wid * POS_PER_W

    tail = lax.convert_element_type(
        lax.rem(lax.iota(jnp.int32, LANES), 2), jnp.float32
    )

    idesc = [
        pltpu.async_copy(
            x_hbm.at[b, pl.ds(pbase, POS_PER_W)],
            idx_v.at[pl.ds(b * POS_PER_W, POS_PER_W)],
            isem,
        )
        for b in range(BATCH)
    ]
    pdesc = pltpu.async_copy(pos_hbm.at[pl.ds(pbase, POS_PER_W)], pos_v, psem)
    for d in idesc:
        d.wait()

    def start_gather(ci):
        return pltpu.async_copy(
            table_hbm.at[idx_v.at[pl.ds(ci * POS_PER_W, CHUNK)]],
            rows[ci & 1],
            gsem[ci & 1],
        )

    gd = {0: start_gather(0)}
    wd = {}
    for ci in range(NCHUNKS):
        buf = ci & 1
        if ci + 1 < NCHUNKS:
            if ci >= 1:
                wd[ci - 1].wait()
            gd[ci + 1] = start_gather(ci + 1)
        gd[ci].wait()
        if ci == 0:
            pdesc.wait()
        rv = rows[buf]

        def add_row(r, _):
            plsc.addupdate(rv.at[r, pl.ds(0, LANES)], pos_v[r, pl.ds(0, LANES)])
            for j in range(1, VECS_PER_ROW):
                plsc.addupdate(rv.at[r, pl.ds(j * LANES, LANES)], tail)
            return _

        lax.fori_loop(0, CHUNK, add_row, None)
        wd[ci] = pltpu.async_copy(
            rv,
            out_hbm.at[pl.ds(ci * SEQ_LEN + pbase, CHUNK)],
            wsem[buf],
        )
    wd[NCHUNKS - 2].wait()
    wd[NCHUNKS - 1].wait()


def kernel(x, table):
    xi = x.astype(jnp.int32)
    out = _embed_sc(xi, table, _POS_HEAD)
    return out.reshape(BATCH, SEQ_LEN, DIM_MODEL)
